# baseline (speedup 1.0000x reference)
import jax
import jax.numpy as jnp
from jax import lax
from jax.experimental import pallas as pl
from jax.experimental.pallas import tpu as pltpu
from jax.experimental.pallas import tpu_sc as plsc

N = 10000
E = 320000
D = 128
NUM_LAB = 11
NC = 2
NS = 16
NW = NC * NS
CHUNK = 50
EPW = E // NW
NCH = EPW // CHUNK
BT = 1000


def _sc_mesh():
    return plsc.VectorSubcoreMesh(core_axis_name="c", subcore_axis_name="s")


DCH = 125
NDCH = EPW // DCH


def _deg_body(dst_hbm, zeros_hbm, out_hbm, dst_v, ones_v, deg_sh, dsem):
    c = lax.axis_index("c")
    s = lax.axis_index("s")
    w = c * NS + s
    for i in range(8):
        ones_v[pl.ds(i * 16, 16)] = jnp.ones((16,), jnp.float32)
    z0 = s * 624
    pltpu.sync_copy(zeros_hbm.at[pl.ds(z0, 640)], deg_sh.at[pl.ds(z0, 640)])
    pltpu.sync_copy(dst_hbm.at[w], dst_v)
    plsc.subcore_barrier()

    def fire(j, carry):
        pltpu.async_copy(ones_v.at[pl.ds(0, DCH)], deg_sh.at[dst_v.at[j]],
                         dsem, add=True)
        return carry

    lax.fori_loop(0, NDCH, fire, 0)

    def drain(j, carry):
        pltpu.make_async_copy(ones_v.at[pl.ds(0, DCH)],
                              deg_sh.at[pl.ds(0, DCH)], dsem).wait()
        return carry

    lax.fori_loop(0, NDCH, drain, 0)
    plsc.subcore_barrier()
    pltpu.sync_copy(deg_sh.at[pl.ds(z0, 640)], out_hbm.at[c, pl.ds(z0, 640)])


def _deg_call(dstd, zeros1):
    return pl.kernel(
        _deg_body,
        out_type=jax.ShapeDtypeStruct((NC, N), jnp.float32),
        mesh=_sc_mesh(),
        compiler_params=pltpu.CompilerParams(use_tc_tiling_on_sc=False),
        scratch_types=[
            pltpu.VMEM((NDCH, DCH), jnp.int32),
            pltpu.VMEM((128,), jnp.float32),
            pltpu.VMEM_SHARED((N,), jnp.float32),
            pltpu.SemaphoreType.DMA,
        ],
    )(dstd, zeros1)


NB = 4


def _seg_body(h_hbm, src_hbm, dst_hbm, zeros_hbm, out_hbm,
              src_v, dst_v, acc_sh, *bufs):
    rows = list(bufs[:NB])
    gsem = list(bufs[NB:2 * NB])
    ssem = list(bufs[2 * NB:])
    c = lax.axis_index("c")
    s = lax.axis_index("s")
    w = c * NS + s
    z0 = s * 625

    def zero_piece(k, carry):
        pltpu.async_copy(zeros_hbm.at[pl.ds(z0 + k * 125, 125)],
                         acc_sh.at[pl.ds(z0 + k * 125, 125)], gsem[0])
        return carry

    lax.fori_loop(0, 5, zero_piece, 0)

    def zero_drain(k, carry):
        pltpu.make_async_copy(zeros_hbm.at[pl.ds(0, 125)],
                              acc_sh.at[pl.ds(0, 125)], gsem[0]).wait()
        return carry

    lax.fori_loop(0, 5, zero_drain, 0)
    pltpu.sync_copy(src_hbm.at[w], src_v)
    pltpu.sync_copy(dst_hbm.at[w], dst_v)
    plsc.subcore_barrier()

    for b in range(NB):
        pltpu.async_copy(h_hbm.at[src_v.at[b]], rows[b], gsem[b])

    def group(g, carry):
        for b in range(NB):
            j = g * NB + b
            pltpu.make_async_copy(
                h_hbm.at[pl.ds(0, CHUNK)], rows[b], gsem[b]).wait()
            pltpu.async_copy(rows[b], acc_sh.at[dst_v.at[j]], ssem[b],
                             add=True)
            pltpu.make_async_copy(
                rows[b], acc_sh.at[pl.ds(0, CHUNK)], ssem[b]).wait()

            @pl.when(j + NB < NCH)
            def _():
                pltpu.async_copy(h_hbm.at[src_v.at[j + NB]], rows[b],
                                 gsem[b])
        return carry

    lax.fori_loop(0, NCH // NB, group, 0)
    plsc.subcore_barrier()

    def out_piece(k, carry):
        pltpu.async_copy(acc_sh.at[pl.ds(z0 + k * 125, 125)],
                         out_hbm.at[c, pl.ds(z0 + k * 125, 125)], gsem[0])
        return carry

    lax.fori_loop(0, 5, out_piece, 0)

    def out_drain(k, carry):
        pltpu.make_async_copy(acc_sh.at[pl.ds(0, 125)],
                              out_hbm.at[c, pl.ds(0, 125)], gsem[0]).wait()
        return carry

    lax.fori_loop(0, 5, out_drain, 0)


def _seg_call(h, src2d, dst2d, zeros2):
    return pl.kernel(
        _seg_body,
        out_type=jax.ShapeDtypeStruct((NC, N, D), jnp.float32),
        mesh=_sc_mesh(),
        compiler_params=pltpu.CompilerParams(use_tc_tiling_on_sc=False),
        scratch_types=[
            pltpu.VMEM((NCH, CHUNK), jnp.int32),
            pltpu.VMEM((NCH, CHUNK), jnp.int32),
            pltpu.VMEM_SHARED((N, D), jnp.float32),
        ] + [pltpu.VMEM((CHUNK, D), jnp.float32)] * NB
          + [pltpu.SemaphoreType.DMA] * (2 * NB),
    )(h, src2d, dst2d, zeros2)


def _in_body(deg_ref, x_ref, w_ref, o_ref, dinv_ref):
    degv = deg_ref[...]
    dinv = lax.rsqrt(jnp.maximum(jnp.sum(degv, axis=1, keepdims=True), 1.0))
    dinv_ref[...] = dinv
    h = jnp.dot(x_ref[...], w_ref[...], preferred_element_type=jnp.float32)
    o_ref[...] = h * dinv


def _mid_body(dinv_ref, sp_ref, w_ref, o_ref):
    dinv = dinv_ref[...]
    sp = sp_ref[...]
    z = jax.nn.relu((sp[0] + sp[1]) * dinv)
    o_ref[...] = jnp.dot(z, w_ref[...],
                         preferred_element_type=jnp.float32) * dinv


def _fin_body(uv_ref, dinv_ref, sp_ref, xlp_ref, wfp_ref, w5c_ref, w1dp_ref,
              w2d_ref, w4r_ref, o_ref, acc, xlu):
    i = pl.program_id(0)

    @pl.when(i == 0)
    def _():
        acc[0] = 0.0
        acc[1] = 0.0
        acc[2] = 0.0
        xlu[...] = jnp.zeros((1, D), jnp.float32)

    u = uv_ref[0]
    v = uv_ref[1]
    dinv = dinv_ref[...]
    sp = sp_ref[...]
    z = jax.nn.relu((sp[0] + sp[1]) * dinv)
    logits = jnp.dot(z, wfp_ref[...], preferred_element_type=jnp.float32)
    xlp = xlp_ref[...]
    col = lax.broadcasted_iota(jnp.int32, (BT, D), 1)
    lm = jnp.where(col < NUM_LAB - 1, logits, -1e30)
    m = jnp.max(lm, axis=1, keepdims=True)
    e = jnp.where(col < NUM_LAB - 1, jnp.exp(logits - m), 0.0)
    den = jnp.sum(e, axis=1, keepdims=True)
    num = jnp.sum(e * xlp, axis=1, keepdims=True)
    acc[0] += jnp.sum(jnp.log(num / den + 1e-12))

    t = jnp.dot(xlp, w5c_ref[...], preferred_element_type=jnp.float32)
    acc[1] += jnp.sum(jnp.exp(t))
    rowid = lax.broadcasted_iota(jnp.int32, (BT, 1), 0) + i * BT
    acc[2] += jnp.sum(jnp.where(rowid == v, t, 0.0))
    xlu[...] += jnp.sum(jnp.where(rowid == u, xlp, 0.0), axis=0,
                        keepdims=True)

    @pl.when(i == pl.num_programs(0) - 1)
    def _():
        q = jax.nn.relu(jnp.dot(xlu[...], w1dp_ref[...],
                                preferred_element_type=jnp.float32))
        r = jax.nn.relu(jnp.dot(q, w2d_ref[...],
                                preferred_element_type=jnp.float32))
        delta = jnp.sum(r * w4r_ref[...])
        tv = acc[2]
        tvd = tv + jnp.where(u == v, delta, 0.0)
        t_stop = w5c_ref[NUM_LAB - 1, 0]
        denom = acc[1] - jnp.exp(tv) + jnp.exp(tvd) + jnp.exp(t_stop)
        res = acc[0] + jnp.log(jnp.exp(tvd) / denom + 1e-12)
        o_ref[...] = jnp.reshape(res, (1, 1))


def _in_call(degp, x, w1):
    return pl.pallas_call(
        _in_body,
        grid=(N // BT,),
        in_specs=[
            pl.BlockSpec((BT, NC), lambda i: (i, 0)),
            pl.BlockSpec((BT, D), lambda i: (i, 0)),
            pl.BlockSpec((D, D), lambda i: (0, 0)),
        ],
        out_specs=[
            pl.BlockSpec((BT, D), lambda i: (i, 0)),
            pl.BlockSpec((BT, 1), lambda i: (i, 0)),
        ],
        out_shape=[
            jax.ShapeDtypeStruct((N, D), jnp.float32),
            jax.ShapeDtypeStruct((N, 1), jnp.float32),
        ],
    )(degp, x, w1)


def _mid_call(dinv, sp, w2):
    return pl.pallas_call(
        _mid_body,
        grid=(N // BT,),
        in_specs=[
            pl.BlockSpec((BT, 1), lambda i: (i, 0)),
            pl.BlockSpec((NC, BT, D), lambda i: (0, i, 0)),
            pl.BlockSpec((D, D), lambda i: (0, 0)),
        ],
        out_specs=pl.BlockSpec((BT, D), lambda i: (i, 0)),
        out_shape=jax.ShapeDtypeStruct((N, D), jnp.float32),
    )(dinv, sp, w2)


def _fin_call(uv, dinv, sp, xlp, wfp, w5c, w1dp, w2d, w4r):
    return pl.pallas_call(
        _fin_body,
        grid=(N // BT,),
        in_specs=[
            pl.BlockSpec(memory_space=pltpu.SMEM),
            pl.BlockSpec((BT, 1), lambda i: (i, 0)),
            pl.BlockSpec((NC, BT, D), lambda i: (0, i, 0)),
            pl.BlockSpec((BT, D), lambda i: (i, 0)),
            pl.BlockSpec((D, D), lambda i: (0, 0)),
            pl.BlockSpec((D, 1), lambda i: (0, 0)),
            pl.BlockSpec((D, D), lambda i: (0, 0)),
            pl.BlockSpec((D, D), lambda i: (0, 0)),
            pl.BlockSpec((1, D), lambda i: (0, 0)),
        ],
        out_specs=pl.BlockSpec((1, 1), lambda i: (0, 0)),
        out_shape=jax.ShapeDtypeStruct((1, 1), jnp.float32),
        scratch_shapes=[
            pltpu.SMEM((4,), jnp.float32),
            pltpu.VMEM((1, D), jnp.float32),
        ],
    )(uv, dinv, sp, xlp, wfp, w5c, w1dp, w2d, w4r)


def kernel(x_p, edge_index_p, x_l, edge_index_l, bfs_index,
           W1_p, W2_p, W1_l, W2_l, W1_d, W2_d, Wf, Wg):
    src2d = edge_index_l[0].reshape(NW, NCH, CHUNK)
    dst2d = edge_index_l[1].reshape(NW, NCH, CHUNK)
    dstd = edge_index_l[1].reshape(NW, NDCH, DCH)
    zeros1 = jnp.zeros((N,), jnp.float32)
    zeros2 = jnp.zeros((N, D), jnp.float32)

    degp = _deg_call(dstd, zeros1)
    h2, dinv = _in_call(degp.T, x_l, W1_l)
    s1 = _seg_call(h2, src2d, dst2d, zeros2)
    h2b = _mid_call(dinv, s1, W2_l)
    s2 = _seg_call(h2b, src2d, dst2d, zeros2)

    xlp = jnp.pad(x_l[:, 4:4 + NUM_LAB], ((0, 0), (0, D - NUM_LAB)))
    wfp = jnp.pad(Wf, ((0, 0), (0, D - NUM_LAB)))
    off = 1 + D + D + NUM_LAB
    w4r = Wg[off:off + D, 0].reshape(1, D)
    w5c = jnp.pad(Wg[off + D:off + D + NUM_LAB, 0],
                  (0, D - NUM_LAB)).reshape(D, 1)
    w1dp = jnp.pad(W1_d, ((0, D - NUM_LAB), (0, 0)))
    uv = bfs_index[0]

    res = _fin_call(uv, dinv, s2, xlp, wfp, w5c, w1dp, W2_d, w4r)
    return res[0, 0]


# speedup vs baseline: 1.0243x; 1.0243x over previous
import jax
import jax.numpy as jnp
from jax import lax
from jax.experimental import pallas as pl
from jax.experimental.pallas import tpu as pltpu
from jax.experimental.pallas import tpu_sc as plsc

N = 10000
E = 320000
D = 128
NUM_LAB = 11
NC = 2
NS = 16
NW = NC * NS
CHUNK = 50
EPW = E // NW
NCH = EPW // CHUNK
BT = 2000


def _sc_mesh():
    return plsc.VectorSubcoreMesh(core_axis_name="c", subcore_axis_name="s")


DCH = 125
NDCH = EPW // DCH


def _deg_body(dst_hbm, zeros_hbm, out_hbm, dst_v, ones_v, deg_sh, dsem):
    c = lax.axis_index("c")
    s = lax.axis_index("s")
    w = c * NS + s
    for i in range(8):
        ones_v[pl.ds(i * 16, 16)] = jnp.ones((16,), jnp.float32)
    z0 = s * 624
    pltpu.sync_copy(zeros_hbm.at[pl.ds(z0, 640)], deg_sh.at[pl.ds(z0, 640)])
    pltpu.sync_copy(dst_hbm.at[w], dst_v)
    plsc.subcore_barrier()

    def fire(j, carry):
        pltpu.async_copy(ones_v.at[pl.ds(0, DCH)], deg_sh.at[dst_v.at[j]],
                         dsem, add=True)
        return carry

    lax.fori_loop(0, NDCH, fire, 0)

    def drain(j, carry):
        pltpu.make_async_copy(ones_v.at[pl.ds(0, DCH)],
                              deg_sh.at[pl.ds(0, DCH)], dsem).wait()
        return carry

    lax.fori_loop(0, NDCH, drain, 0)
    plsc.subcore_barrier()
    pltpu.sync_copy(deg_sh.at[pl.ds(z0, 640)], out_hbm.at[c, pl.ds(z0, 640)])


def _deg_call(dstd, zeros1):
    return pl.kernel(
        _deg_body,
        out_type=jax.ShapeDtypeStruct((NC, N), jnp.float32),
        mesh=_sc_mesh(),
        compiler_params=pltpu.CompilerParams(use_tc_tiling_on_sc=False),
        scratch_types=[
            pltpu.VMEM((NDCH, DCH), jnp.int32),
            pltpu.VMEM((128,), jnp.float32),
            pltpu.VMEM_SHARED((N,), jnp.float32),
            pltpu.SemaphoreType.DMA,
        ],
    )(dstd, zeros1)


NB = 4


def _seg_body(h_hbm, src_hbm, dst_hbm, zeros_hbm, out_hbm,
              src_v, dst_v, acc_sh, *bufs):
    rows = list(bufs[:NB])
    gsem = list(bufs[NB:2 * NB])
    ssem = list(bufs[2 * NB:])
    c = lax.axis_index("c")
    s = lax.axis_index("s")
    w = c * NS + s
    z0 = s * 625

    def zero_piece(k, carry):
        pltpu.async_copy(zeros_hbm.at[pl.ds(z0 + k * 125, 125)],
                         acc_sh.at[pl.ds(z0 + k * 125, 125)], gsem[0])
        return carry

    lax.fori_loop(0, 5, zero_piece, 0)

    def zero_drain(k, carry):
        pltpu.make_async_copy(zeros_hbm.at[pl.ds(0, 125)],
                              acc_sh.at[pl.ds(0, 125)], gsem[0]).wait()
        return carry

    lax.fori_loop(0, 5, zero_drain, 0)
    pltpu.sync_copy(src_hbm.at[w], src_v)
    pltpu.sync_copy(dst_hbm.at[w], dst_v)
    plsc.subcore_barrier()

    for b in range(NB):
        pltpu.async_copy(h_hbm.at[src_v.at[b]], rows[b], gsem[b])

    def group(g, carry):
        for b in range(NB):
            j = g * NB + b
            pltpu.make_async_copy(
                h_hbm.at[pl.ds(0, CHUNK)], rows[b], gsem[b]).wait()
            pltpu.async_copy(rows[b], acc_sh.at[dst_v.at[j]], ssem[b],
                             add=True)
            pltpu.make_async_copy(
                rows[b], acc_sh.at[pl.ds(0, CHUNK)], ssem[b]).wait()

            @pl.when(j + NB < NCH)
            def _():
                pltpu.async_copy(h_hbm.at[src_v.at[j + NB]], rows[b],
                                 gsem[b])
        return carry

    lax.fori_loop(0, NCH // NB, group, 0)
    plsc.subcore_barrier()

    def out_piece(k, carry):
        pltpu.async_copy(acc_sh.at[pl.ds(z0 + k * 125, 125)],
                         out_hbm.at[c, pl.ds(z0 + k * 125, 125)], gsem[0])
        return carry

    lax.fori_loop(0, 5, out_piece, 0)

    def out_drain(k, carry):
        pltpu.make_async_copy(acc_sh.at[pl.ds(0, 125)],
                              out_hbm.at[c, pl.ds(0, 125)], gsem[0]).wait()
        return carry

    lax.fori_loop(0, 5, out_drain, 0)


def _seg_call(h, src2d, dst2d, zeros2):
    return pl.kernel(
        _seg_body,
        out_type=jax.ShapeDtypeStruct((NC, N, D), jnp.float32),
        mesh=_sc_mesh(),
        compiler_params=pltpu.CompilerParams(use_tc_tiling_on_sc=False),
        scratch_types=[
            pltpu.VMEM((NCH, CHUNK), jnp.int32),
            pltpu.VMEM((NCH, CHUNK), jnp.int32),
            pltpu.VMEM_SHARED((N, D), jnp.float32),
        ] + [pltpu.VMEM((CHUNK, D), jnp.float32)] * NB
          + [pltpu.SemaphoreType.DMA] * (2 * NB),
    )(h, src2d, dst2d, zeros2)


def _in_body(deg_ref, x_ref, w_ref, o_ref, dinv_ref):
    degv = deg_ref[...]
    dinv = lax.rsqrt(jnp.maximum(jnp.sum(degv, axis=1, keepdims=True), 1.0))
    dinv_ref[...] = dinv
    h = jnp.dot(x_ref[...], w_ref[...], preferred_element_type=jnp.float32)
    o_ref[...] = h * dinv


def _mid_body(dinv_ref, sp_ref, w_ref, o_ref):
    dinv = dinv_ref[...]
    sp = sp_ref[...]
    z = jax.nn.relu((sp[0] + sp[1]) * dinv)
    o_ref[...] = jnp.dot(z, w_ref[...],
                         preferred_element_type=jnp.float32) * dinv


def _fin_body(uv_ref, dinv_ref, sp_ref, xlp_ref, wfp_ref, w5c_ref, w1dp_ref,
              w2d_ref, w4r_ref, o_ref, acc, xlu):
    i = pl.program_id(0)

    @pl.when(i == 0)
    def _():
        acc[0] = 0.0
        acc[1] = 0.0
        acc[2] = 0.0
        xlu[...] = jnp.zeros((1, D), jnp.float32)

    u = uv_ref[0]
    v = uv_ref[1]
    dinv = dinv_ref[...]
    sp = sp_ref[...]
    z = jax.nn.relu((sp[0] + sp[1]) * dinv)
    logits = jnp.dot(z, wfp_ref[...], preferred_element_type=jnp.float32)
    xlp = xlp_ref[...]
    col = lax.broadcasted_iota(jnp.int32, (BT, D), 1)
    lm = jnp.where(col < NUM_LAB - 1, logits, -1e30)
    m = jnp.max(lm, axis=1, keepdims=True)
    e = jnp.where(col < NUM_LAB - 1, jnp.exp(logits - m), 0.0)
    den = jnp.sum(e, axis=1, keepdims=True)
    num = jnp.sum(e * xlp, axis=1, keepdims=True)
    acc[0] += jnp.sum(jnp.log(num / den + 1e-12))

    t = jnp.dot(xlp, w5c_ref[...], preferred_element_type=jnp.float32)
    acc[1] += jnp.sum(jnp.exp(t))
    rowid = lax.broadcasted_iota(jnp.int32, (BT, 1), 0) + i * BT
    acc[2] += jnp.sum(jnp.where(rowid == v, t, 0.0))
    xlu[...] += jnp.sum(jnp.where(rowid == u, xlp, 0.0), axis=0,
                        keepdims=True)

    @pl.when(i == pl.num_programs(0) - 1)
    def _():
        q = jax.nn.relu(jnp.dot(xlu[...], w1dp_ref[...],
                                preferred_element_type=jnp.float32))
        r = jax.nn.relu(jnp.dot(q, w2d_ref[...],
                                preferred_element_type=jnp.float32))
        delta = jnp.sum(r * w4r_ref[...])
        tv = acc[2]
        tvd = tv + jnp.where(u == v, delta, 0.0)
        t_stop = w5c_ref[NUM_LAB - 1, 0]
        denom = acc[1] - jnp.exp(tv) + jnp.exp(tvd) + jnp.exp(t_stop)
        res = acc[0] + jnp.log(jnp.exp(tvd) / denom + 1e-12)
        o_ref[...] = jnp.reshape(res, (1, 1))


def _in_call(degp, x, w1):
    return pl.pallas_call(
        _in_body,
        grid=(N // BT,),
        in_specs=[
            pl.BlockSpec((BT, NC), lambda i: (i, 0)),
            pl.BlockSpec((BT, D), lambda i: (i, 0)),
            pl.BlockSpec((D, D), lambda i: (0, 0)),
        ],
        out_specs=[
            pl.BlockSpec((BT, D), lambda i: (i, 0)),
            pl.BlockSpec((BT, 1), lambda i: (i, 0)),
        ],
        out_shape=[
            jax.ShapeDtypeStruct((N, D), jnp.float32),
            jax.ShapeDtypeStruct((N, 1), jnp.float32),
        ],
    )(degp, x, w1)


def _mid_call(dinv, sp, w2):
    return pl.pallas_call(
        _mid_body,
        grid=(N // BT,),
        in_specs=[
            pl.BlockSpec((BT, 1), lambda i: (i, 0)),
            pl.BlockSpec((NC, BT, D), lambda i: (0, i, 0)),
            pl.BlockSpec((D, D), lambda i: (0, 0)),
        ],
        out_specs=pl.BlockSpec((BT, D), lambda i: (i, 0)),
        out_shape=jax.ShapeDtypeStruct((N, D), jnp.float32),
    )(dinv, sp, w2)


def _fin_call(uv, dinv, sp, xlp, wfp, w5c, w1dp, w2d, w4r):
    return pl.pallas_call(
        _fin_body,
        grid=(N // BT,),
        in_specs=[
            pl.BlockSpec(memory_space=pltpu.SMEM),
            pl.BlockSpec((BT, 1), lambda i: (i, 0)),
            pl.BlockSpec((NC, BT, D), lambda i: (0, i, 0)),
            pl.BlockSpec((BT, D), lambda i: (i, 0)),
            pl.BlockSpec((D, D), lambda i: (0, 0)),
            pl.BlockSpec((D, 1), lambda i: (0, 0)),
            pl.BlockSpec((D, D), lambda i: (0, 0)),
            pl.BlockSpec((D, D), lambda i: (0, 0)),
            pl.BlockSpec((1, D), lambda i: (0, 0)),
        ],
        out_specs=pl.BlockSpec((1, 1), lambda i: (0, 0)),
        out_shape=jax.ShapeDtypeStruct((1, 1), jnp.float32),
        scratch_shapes=[
            pltpu.SMEM((4,), jnp.float32),
            pltpu.VMEM((1, D), jnp.float32),
        ],
    )(uv, dinv, sp, xlp, wfp, w5c, w1dp, w2d, w4r)


def kernel(x_p, edge_index_p, x_l, edge_index_l, bfs_index,
           W1_p, W2_p, W1_l, W2_l, W1_d, W2_d, Wf, Wg):
    src2d = edge_index_l[0].reshape(NW, NCH, CHUNK)
    dst2d = edge_index_l[1].reshape(NW, NCH, CHUNK)
    dstd = edge_index_l[1].reshape(NW, NDCH, DCH)
    zeros1 = jnp.zeros((N,), jnp.float32)
    zeros2 = jnp.zeros((N, D), jnp.float32)

    degp = _deg_call(dstd, zeros1)
    h2, dinv = _in_call(degp.T, x_l, W1_l)
    s1 = _seg_call(h2, src2d, dst2d, zeros2)
    h2b = _mid_call(dinv, s1, W2_l)
    s2 = _seg_call(h2b, src2d, dst2d, zeros2)

    xlp = jnp.pad(x_l[:, 4:4 + NUM_LAB], ((0, 0), (0, D - NUM_LAB)))
    wfp = jnp.pad(Wf, ((0, 0), (0, D - NUM_LAB)))
    off = 1 + D + D + NUM_LAB
    w4r = Wg[off:off + D, 0].reshape(1, D)
    w5c = jnp.pad(Wg[off + D:off + D + NUM_LAB, 0],
                  (0, D - NUM_LAB)).reshape(D, 1)
    w1dp = jnp.pad(W1_d, ((0, D - NUM_LAB), (0, 0)))
    uv = bfs_index[0]

    res = _fin_call(uv, dinv, s2, xlp, wfp, w5c, w1dp, W2_d, w4r)
    return res[0, 0]


# speedup vs baseline: 1.0391x; 1.0145x over previous
import jax
import jax.numpy as jnp
from jax import lax
from jax.experimental import pallas as pl
from jax.experimental.pallas import tpu as pltpu
from jax.experimental.pallas import tpu_sc as plsc

N = 10000
E = 320000
D = 128
NUM_LAB = 11
NC = 2
NS = 16
NW = NC * NS
CHUNK = 50
EPW = E // NW
NCH = EPW // CHUNK
BT = 5000


def _sc_mesh():
    return plsc.VectorSubcoreMesh(core_axis_name="c", subcore_axis_name="s")


DCH = 125
NDCH = EPW // DCH


def _deg_body(dst_hbm, zeros_hbm, out_hbm, dst_v, ones_v, deg_sh, dsem):
    c = lax.axis_index("c")
    s = lax.axis_index("s")
    w = c * NS + s
    for i in range(8):
        ones_v[pl.ds(i * 16, 16)] = jnp.ones((16,), jnp.float32)
    z0 = s * 624
    pltpu.sync_copy(zeros_hbm.at[pl.ds(z0, 640)], deg_sh.at[pl.ds(z0, 640)])
    pltpu.sync_copy(dst_hbm.at[w], dst_v)
    plsc.subcore_barrier()

    def fire(j, carry):
        pltpu.async_copy(ones_v.at[pl.ds(0, DCH)], deg_sh.at[dst_v.at[j]],
                         dsem, add=True)
        return carry

    lax.fori_loop(0, NDCH, fire, 0)

    def drain(j, carry):
        pltpu.make_async_copy(ones_v.at[pl.ds(0, DCH)],
                              deg_sh.at[pl.ds(0, DCH)], dsem).wait()
        return carry

    lax.fori_loop(0, NDCH, drain, 0)
    plsc.subcore_barrier()
    pltpu.sync_copy(deg_sh.at[pl.ds(z0, 640)], out_hbm.at[c, pl.ds(z0, 640)])


def _deg_call(dstd, zeros1):
    return pl.kernel(
        _deg_body,
        out_type=jax.ShapeDtypeStruct((NC, N), jnp.float32),
        mesh=_sc_mesh(),
        compiler_params=pltpu.CompilerParams(use_tc_tiling_on_sc=False),
        scratch_types=[
            pltpu.VMEM((NDCH, DCH), jnp.int32),
            pltpu.VMEM((128,), jnp.float32),
            pltpu.VMEM_SHARED((N,), jnp.float32),
            pltpu.SemaphoreType.DMA,
        ],
    )(dstd, zeros1)


NB = 4


def _seg_body(h_hbm, src_hbm, dst_hbm, zeros_hbm, out_hbm,
              src_v, dst_v, acc_sh, *bufs):
    rows = list(bufs[:NB])
    gsem = list(bufs[NB:2 * NB])
    ssem = list(bufs[2 * NB:])
    c = lax.axis_index("c")
    s = lax.axis_index("s")
    w = c * NS + s
    z0 = s * 625

    def zero_piece(k, carry):
        pltpu.async_copy(zeros_hbm.at[pl.ds(z0 + k * 125, 125)],
                         acc_sh.at[pl.ds(z0 + k * 125, 125)], gsem[0])
        return carry

    lax.fori_loop(0, 5, zero_piece, 0)

    def zero_drain(k, carry):
        pltpu.make_async_copy(zeros_hbm.at[pl.ds(0, 125)],
                              acc_sh.at[pl.ds(0, 125)], gsem[0]).wait()
        return carry

    lax.fori_loop(0, 5, zero_drain, 0)
    pltpu.sync_copy(src_hbm.at[w], src_v)
    pltpu.sync_copy(dst_hbm.at[w], dst_v)
    plsc.subcore_barrier()

    for b in range(NB):
        pltpu.async_copy(h_hbm.at[src_v.at[b]], rows[b], gsem[b])

    def group(g, carry):
        for b in range(NB):
            j = g * NB + b
            pltpu.make_async_copy(
                h_hbm.at[pl.ds(0, CHUNK)], rows[b], gsem[b]).wait()
            pltpu.async_copy(rows[b], acc_sh.at[dst_v.at[j]], ssem[b],
                             add=True)
            pltpu.make_async_copy(
                rows[b], acc_sh.at[pl.ds(0, CHUNK)], ssem[b]).wait()

            @pl.when(j + NB < NCH)
            def _():
                pltpu.async_copy(h_hbm.at[src_v.at[j + NB]], rows[b],
                                 gsem[b])
        return carry

    lax.fori_loop(0, NCH // NB, group, 0)
    plsc.subcore_barrier()

    def out_piece(k, carry):
        pltpu.async_copy(acc_sh.at[pl.ds(z0 + k * 125, 125)],
                         out_hbm.at[c, pl.ds(z0 + k * 125, 125)], gsem[0])
        return carry

    lax.fori_loop(0, 5, out_piece, 0)

    def out_drain(k, carry):
        pltpu.make_async_copy(acc_sh.at[pl.ds(0, 125)],
                              out_hbm.at[c, pl.ds(0, 125)], gsem[0]).wait()
        return carry

    lax.fori_loop(0, 5, out_drain, 0)


def _seg_call(h, src2d, dst2d, zeros2):
    return pl.kernel(
        _seg_body,
        out_type=jax.ShapeDtypeStruct((NC, N, D), jnp.float32),
        mesh=_sc_mesh(),
        compiler_params=pltpu.CompilerParams(use_tc_tiling_on_sc=False),
        scratch_types=[
            pltpu.VMEM((NCH, CHUNK), jnp.int32),
            pltpu.VMEM((NCH, CHUNK), jnp.int32),
            pltpu.VMEM_SHARED((N, D), jnp.float32),
        ] + [pltpu.VMEM((CHUNK, D), jnp.float32)] * NB
          + [pltpu.SemaphoreType.DMA] * (2 * NB),
    )(h, src2d, dst2d, zeros2)


def _in_body(deg_ref, x_ref, w_ref, o_ref, dinv_ref):
    degv = deg_ref[...]
    dinv = lax.rsqrt(jnp.maximum(jnp.sum(degv, axis=1, keepdims=True), 1.0))
    dinv_ref[...] = dinv
    h = jnp.dot(x_ref[...], w_ref[...], preferred_element_type=jnp.float32)
    o_ref[...] = h * dinv


def _mid_body(dinv_ref, sp_ref, w_ref, o_ref):
    dinv = dinv_ref[...]
    sp = sp_ref[...]
    z = jax.nn.relu((sp[0] + sp[1]) * dinv)
    o_ref[...] = jnp.dot(z, w_ref[...],
                         preferred_element_type=jnp.float32) * dinv


def _fin_body(uv_ref, dinv_ref, sp_ref, xlp_ref, wfp_ref, w5c_ref, w1dp_ref,
              w2d_ref, w4r_ref, o_ref, acc, xlu):
    i = pl.program_id(0)

    @pl.when(i == 0)
    def _():
        acc[0] = 0.0
        acc[1] = 0.0
        acc[2] = 0.0
        xlu[...] = jnp.zeros((1, D), jnp.float32)

    u = uv_ref[0]
    v = uv_ref[1]
    dinv = dinv_ref[...]
    sp = sp_ref[...]
    z = jax.nn.relu((sp[0] + sp[1]) * dinv)
    logits = jnp.dot(z, wfp_ref[...], preferred_element_type=jnp.float32)
    xlp = xlp_ref[...]
    col = lax.broadcasted_iota(jnp.int32, (BT, D), 1)
    lm = jnp.where(col < NUM_LAB - 1, logits, -1e30)
    m = jnp.max(lm, axis=1, keepdims=True)
    e = jnp.where(col < NUM_LAB - 1, jnp.exp(logits - m), 0.0)
    den = jnp.sum(e, axis=1, keepdims=True)
    num = jnp.sum(e * xlp, axis=1, keepdims=True)
    acc[0] += jnp.sum(jnp.log(num / den + 1e-12))

    t = jnp.dot(xlp, w5c_ref[...], preferred_element_type=jnp.float32)
    acc[1] += jnp.sum(jnp.exp(t))
    rowid = lax.broadcasted_iota(jnp.int32, (BT, 1), 0) + i * BT
    acc[2] += jnp.sum(jnp.where(rowid == v, t, 0.0))
    xlu[...] += jnp.sum(jnp.where(rowid == u, xlp, 0.0), axis=0,
                        keepdims=True)

    @pl.when(i == pl.num_programs(0) - 1)
    def _():
        q = jax.nn.relu(jnp.dot(xlu[...], w1dp_ref[...],
                                preferred_element_type=jnp.float32))
        r = jax.nn.relu(jnp.dot(q, w2d_ref[...],
                                preferred_element_type=jnp.float32))
        delta = jnp.sum(r * w4r_ref[...])
        tv = acc[2]
        tvd = tv + jnp.where(u == v, delta, 0.0)
        t_stop = w5c_ref[NUM_LAB - 1, 0]
        denom = acc[1] - jnp.exp(tv) + jnp.exp(tvd) + jnp.exp(t_stop)
        res = acc[0] + jnp.log(jnp.exp(tvd) / denom + 1e-12)
        o_ref[...] = jnp.reshape(res, (1, 1))


def _in_call(degp, x, w1):
    return pl.pallas_call(
        _in_body,
        grid=(N // BT,),
        in_specs=[
            pl.BlockSpec((BT, NC), lambda i: (i, 0)),
            pl.BlockSpec((BT, D), lambda i: (i, 0)),
            pl.BlockSpec((D, D), lambda i: (0, 0)),
        ],
        out_specs=[
            pl.BlockSpec((BT, D), lambda i: (i, 0)),
            pl.BlockSpec((BT, 1), lambda i: (i, 0)),
        ],
        out_shape=[
            jax.ShapeDtypeStruct((N, D), jnp.float32),
            jax.ShapeDtypeStruct((N, 1), jnp.float32),
        ],
    )(degp, x, w1)


def _mid_call(dinv, sp, w2):
    return pl.pallas_call(
        _mid_body,
        grid=(N // BT,),
        in_specs=[
            pl.BlockSpec((BT, 1), lambda i: (i, 0)),
            pl.BlockSpec((NC, BT, D), lambda i: (0, i, 0)),
            pl.BlockSpec((D, D), lambda i: (0, 0)),
        ],
        out_specs=pl.BlockSpec((BT, D), lambda i: (i, 0)),
        out_shape=jax.ShapeDtypeStruct((N, D), jnp.float32),
    )(dinv, sp, w2)


def _fin_call(uv, dinv, sp, xlp, wfp, w5c, w1dp, w2d, w4r):
    return pl.pallas_call(
        _fin_body,
        grid=(N // BT,),
        in_specs=[
            pl.BlockSpec(memory_space=pltpu.SMEM),
            pl.BlockSpec((BT, 1), lambda i: (i, 0)),
            pl.BlockSpec((NC, BT, D), lambda i: (0, i, 0)),
            pl.BlockSpec((BT, D), lambda i: (i, 0)),
            pl.BlockSpec((D, D), lambda i: (0, 0)),
            pl.BlockSpec((D, 1), lambda i: (0, 0)),
            pl.BlockSpec((D, D), lambda i: (0, 0)),
            pl.BlockSpec((D, D), lambda i: (0, 0)),
            pl.BlockSpec((1, D), lambda i: (0, 0)),
        ],
        out_specs=pl.BlockSpec((1, 1), lambda i: (0, 0)),
        out_shape=jax.ShapeDtypeStruct((1, 1), jnp.float32),
        scratch_shapes=[
            pltpu.SMEM((4,), jnp.float32),
            pltpu.VMEM((1, D), jnp.float32),
        ],
    )(uv, dinv, sp, xlp, wfp, w5c, w1dp, w2d, w4r)


def kernel(x_p, edge_index_p, x_l, edge_index_l, bfs_index,
           W1_p, W2_p, W1_l, W2_l, W1_d, W2_d, Wf, Wg):
    src2d = edge_index_l[0].reshape(NW, NCH, CHUNK)
    dst2d = edge_index_l[1].reshape(NW, NCH, CHUNK)
    dstd = edge_index_l[1].reshape(NW, NDCH, DCH)
    zeros1 = jnp.zeros((N,), jnp.float32)
    zeros2 = jnp.zeros((N, D), jnp.float32)

    degp = _deg_call(dstd, zeros1)
    h2, dinv = _in_call(degp.T, x_l, W1_l)
    s1 = _seg_call(h2, src2d, dst2d, zeros2)
    h2b = _mid_call(dinv, s1, W2_l)
    s2 = _seg_call(h2b, src2d, dst2d, zeros2)

    xlp = jnp.pad(x_l[:, 4:4 + NUM_LAB], ((0, 0), (0, D - NUM_LAB)))
    wfp = jnp.pad(Wf, ((0, 0), (0, D - NUM_LAB)))
    off = 1 + D + D + NUM_LAB
    w4r = Wg[off:off + D, 0].reshape(1, D)
    w5c = jnp.pad(Wg[off + D:off + D + NUM_LAB, 0],
                  (0, D - NUM_LAB)).reshape(D, 1)
    w1dp = jnp.pad(W1_d, ((0, D - NUM_LAB), (0, 0)))
    uv = bfs_index[0]

    res = _fin_call(uv, dinv, s2, xlp, wfp, w5c, w1dp, W2_d, w4r)
    return res[0, 0]
